# Initial kernel scaffold; baseline (speedup 1.0000x reference)
import jax
import jax.numpy as jnp
from jax.experimental import pallas as pl


def kernel(x, adj, gc1_w, gc2_w, gc3_w, lin1_w, lin1_b, lin3_w, lin3_b):
    raise NotImplementedError("write your pallas kernel here")


# speedup vs baseline: 1.3893x; 1.3893x over previous
import jax
import jax.numpy as jnp
from jax.experimental import pallas as pl
from jax.experimental.pallas import tpu as pltpu

_BM = 200


def _pass1_body(x_ref, adj_ref, w13_ref, l1wt_ref, l1b_ref,
                p_ref, hl_ref, s_ref):
    i = pl.program_id(0)

    @pl.when(i == 0)
    def _():
        s_ref[...] = jnp.dot(x_ref[...], w13_ref[...],
                             preferred_element_type=jnp.float32)

    g = jnp.dot(adj_ref[...], s_ref[...], preferred_element_type=jnp.float32)
    p_ref[...] = jnp.maximum(g, 0.0)
    xb = x_ref[pl.ds(i * _BM, _BM), :]
    hl = jnp.dot(xb, l1wt_ref[...], preferred_element_type=jnp.float32)
    hl_ref[...] = jnp.maximum(hl + l1b_ref[...], 0.0)


def _pass2_body(p_ref, hl_ref, adj_ref, gc2_ref, l3wt_ref, l3b_ref,
                z_ref, t_ref):
    i = pl.program_id(0)

    @pl.when(i == 0)
    def _():
        t_ref[...] = jnp.dot(p_ref[:, 0:64], gc2_ref[...],
                             preferred_element_type=jnp.float32)

    acc = jnp.dot(adj_ref[...], t_ref[...], preferred_element_type=jnp.float32)
    hg1 = jnp.maximum(acc, 0.0)
    hg2 = p_ref[pl.ds(i * _BM, _BM), 64:128]
    hl = hl_ref[...]
    z = (jnp.dot(hg1, l3wt_ref[0:64, :], preferred_element_type=jnp.float32)
         + jnp.dot(hg2, l3wt_ref[64:128, :], preferred_element_type=jnp.float32)
         + jnp.dot(hl, l3wt_ref[128:192, :], preferred_element_type=jnp.float32)
         + l3b_ref[...])
    z_ref[...] = z


def kernel(x, adj, gc1_w, gc2_w, gc3_w, lin1_w, lin1_b, lin3_w, lin3_b):
    N, F = x.shape
    H = gc1_w.shape[1]
    w13 = jnp.concatenate([gc1_w, gc3_w], axis=1)
    l1wt = lin1_w.T
    l1b = lin1_b.reshape(1, H)
    l3wt = lin3_w.T
    l3b = lin3_b.reshape(1, H)
    grid = (N // _BM,)

    p, hl = pl.pallas_call(
        _pass1_body,
        grid=grid,
        in_specs=[
            pl.BlockSpec((N, F), lambda i: (0, 0)),
            pl.BlockSpec((_BM, N), lambda i: (i, 0)),
            pl.BlockSpec((F, 2 * H), lambda i: (0, 0)),
            pl.BlockSpec((F, H), lambda i: (0, 0)),
            pl.BlockSpec((1, H), lambda i: (0, 0)),
        ],
        out_specs=[
            pl.BlockSpec((_BM, 2 * H), lambda i: (i, 0)),
            pl.BlockSpec((_BM, H), lambda i: (i, 0)),
        ],
        out_shape=[
            jax.ShapeDtypeStruct((N, 2 * H), jnp.float32),
            jax.ShapeDtypeStruct((N, H), jnp.float32),
        ],
        scratch_shapes=[pltpu.VMEM((N, 2 * H), jnp.float32)],
    )(x, adj, w13, l1wt, l1b)

    z = pl.pallas_call(
        _pass2_body,
        grid=grid,
        in_specs=[
            pl.BlockSpec((N, 2 * H), lambda i: (0, 0)),
            pl.BlockSpec((_BM, H), lambda i: (i, 0)),
            pl.BlockSpec((_BM, N), lambda i: (i, 0)),
            pl.BlockSpec((H, H), lambda i: (0, 0)),
            pl.BlockSpec((3 * H, H), lambda i: (0, 0)),
            pl.BlockSpec((1, H), lambda i: (0, 0)),
        ],
        out_specs=pl.BlockSpec((_BM, H), lambda i: (i, 0)),
        out_shape=jax.ShapeDtypeStruct((N, H), jnp.float32),
        scratch_shapes=[pltpu.VMEM((N, H), jnp.float32)],
    )(p, hl, adj, gc2_w, l3wt, l3b)

    return (z, z)


# speedup vs baseline: 1.4207x; 1.0226x over previous
import jax
import jax.numpy as jnp
from jax.experimental import pallas as pl
from jax.experimental.pallas import tpu as pltpu

_BM = 400


def _pass1_body(x_ref, adj_ref, w13_ref, l1wt_ref, l1b_ref,
                p_ref, hl_ref, s_ref):
    i = pl.program_id(0)

    @pl.when(i == 0)
    def _():
        s_ref[...] = jnp.dot(x_ref[...], w13_ref[...],
                             preferred_element_type=jnp.float32)

    g = jnp.dot(adj_ref[...], s_ref[...], preferred_element_type=jnp.float32)
    p_ref[...] = jnp.maximum(g, 0.0)
    xb = x_ref[pl.ds(i * _BM, _BM), :]
    hl = jnp.dot(xb, l1wt_ref[...], preferred_element_type=jnp.float32)
    hl_ref[...] = jnp.maximum(hl + l1b_ref[...], 0.0)


def _pass2_body(p_ref, hl_ref, adj_ref, gc2_ref, l3wt_ref, l3b_ref,
                z_ref, t_ref):
    i = pl.program_id(0)

    @pl.when(i == 0)
    def _():
        t_ref[...] = jnp.dot(p_ref[:, 0:64], gc2_ref[...],
                             preferred_element_type=jnp.float32)

    acc = jnp.dot(adj_ref[...], t_ref[...], preferred_element_type=jnp.float32)
    hg1 = jnp.maximum(acc, 0.0)
    hg2 = p_ref[pl.ds(i * _BM, _BM), 64:128]
    hl = hl_ref[...]
    z = (jnp.dot(hg1, l3wt_ref[0:64, :], preferred_element_type=jnp.float32)
         + jnp.dot(hg2, l3wt_ref[64:128, :], preferred_element_type=jnp.float32)
         + jnp.dot(hl, l3wt_ref[128:192, :], preferred_element_type=jnp.float32)
         + l3b_ref[...])
    z_ref[...] = z


def kernel(x, adj, gc1_w, gc2_w, gc3_w, lin1_w, lin1_b, lin3_w, lin3_b):
    N, F = x.shape
    H = gc1_w.shape[1]
    w13 = jnp.concatenate([gc1_w, gc3_w], axis=1)
    l1wt = lin1_w.T
    l1b = lin1_b.reshape(1, H)
    l3wt = lin3_w.T
    l3b = lin3_b.reshape(1, H)
    grid = (N // _BM,)

    p, hl = pl.pallas_call(
        _pass1_body,
        grid=grid,
        in_specs=[
            pl.BlockSpec((N, F), lambda i: (0, 0)),
            pl.BlockSpec((_BM, N), lambda i: (i, 0)),
            pl.BlockSpec((F, 2 * H), lambda i: (0, 0)),
            pl.BlockSpec((F, H), lambda i: (0, 0)),
            pl.BlockSpec((1, H), lambda i: (0, 0)),
        ],
        out_specs=[
            pl.BlockSpec((_BM, 2 * H), lambda i: (i, 0)),
            pl.BlockSpec((_BM, H), lambda i: (i, 0)),
        ],
        out_shape=[
            jax.ShapeDtypeStruct((N, 2 * H), jnp.float32),
            jax.ShapeDtypeStruct((N, H), jnp.float32),
        ],
        scratch_shapes=[pltpu.VMEM((N, 2 * H), jnp.float32)],
    )(x, adj, w13, l1wt, l1b)

    z = pl.pallas_call(
        _pass2_body,
        grid=grid,
        in_specs=[
            pl.BlockSpec((N, 2 * H), lambda i: (0, 0)),
            pl.BlockSpec((_BM, H), lambda i: (i, 0)),
            pl.BlockSpec((_BM, N), lambda i: (i, 0)),
            pl.BlockSpec((H, H), lambda i: (0, 0)),
            pl.BlockSpec((3 * H, H), lambda i: (0, 0)),
            pl.BlockSpec((1, H), lambda i: (0, 0)),
        ],
        out_specs=pl.BlockSpec((_BM, H), lambda i: (i, 0)),
        out_shape=jax.ShapeDtypeStruct((N, H), jnp.float32),
        scratch_shapes=[pltpu.VMEM((N, H), jnp.float32)],
    )(p, hl, adj, gc2_w, l3wt, l3b)

    return (z, z)
